# baseline (speedup 1.0000x reference)
import functools

import jax
import jax.numpy as jnp
from jax import lax
from jax.experimental import pallas as pl
from jax.experimental.pallas import tpu as pltpu
from jax.experimental.pallas import tpu_sc as plsc

N_NODES = 10000
N_EDGES = 320000
EMB_DIM = 128
ATOM_VOCAB = 137
BOND_VOCAB = 14
N_ATOM_FEATS = 9
N_BOND_FEATS = 3

NC, NS = 2, 16
NW = NC * NS

ATOM_ROWS = N_ATOM_FEATS * ATOM_VOCAB
ATOM_ROWS_PAD = 1280
ATOM_STAGE = ATOM_ROWS_PAD // NS

COMBO_ROWS = BOND_VOCAB ** 3
COMBO_ROWS_PAD = 2816
COMBO_PER_SUB = COMBO_ROWS_PAD // NS
COMBO_CHUNK = COMBO_PER_SUB // 2

EB = 128
E_BLOCKS = N_EDGES // EB
KE = 80

NB = 16
N_BLOCKS = N_NODES // NB
KN = 20
NIDX = N_ATOM_FEATS * NB


def _sc_body(xt, et, atomf, bondf, node_out, edge_out,
             bond_v, abuf, ei, cidx, ebuf, nidx, nbufs, combo_sp, atom_sp,
             gsems, wsems, nsems, nwsems, isems, nisems):
    c = lax.axis_index("c")
    s = lax.axis_index("s")
    wid = s * NC + c

    pltpu.sync_copy(bondf, bond_v)

    a0 = s * ATOM_STAGE
    pltpu.sync_copy(atomf.at[pl.ds(a0, ATOM_STAGE)], abuf.at[pl.ds(0, ATOM_STAGE)])
    pltpu.sync_copy(abuf.at[pl.ds(0, ATOM_STAGE)], atom_sp.at[pl.ds(a0, ATOM_STAGE)])

    for half in range(2):
        r0 = s * COMBO_PER_SUB + half * COMBO_CHUNK

        def build_row(i, _, r0=r0):
            r = r0 + i
            j0 = r // (BOND_VOCAB * BOND_VOCAB)
            j1 = (r // BOND_VOCAB) % BOND_VOCAB
            j2 = r % BOND_VOCAB
            for cc in range(EMB_DIM // 16):
                sl = pl.ds(cc * 16, 16)
                abuf[i, sl] = (bond_v[j0, sl]
                               + bond_v[BOND_VOCAB + j1, sl]
                               + bond_v[2 * BOND_VOCAB + j2, sl])
            return _

        lax.fori_loop(0, COMBO_CHUNK, build_row, 0)
        pltpu.sync_copy(abuf, combo_sp.at[pl.ds(r0, COMBO_CHUNK)])

    plsc.subcore_barrier()

    def eblk(k):
        return jnp.minimum(wid + k * NW, E_BLOCKS - 1)

    def e_load_start(k, r):
        pltpu.async_copy(et.at[pl.ds(eblk(k) * (N_BOND_FEATS * EB),
                                     N_BOND_FEATS * EB)], ei[r], isems[r])

    def e_load_wait(k, r):
        pltpu.make_async_copy(et.at[pl.ds(eblk(k) * (N_BOND_FEATS * EB),
                                          N_BOND_FEATS * EB)], ei[r],
                              isems[r]).wait()

    def e_cidx(k, r):
        for i in range(EB // 16):
            cidx[r][pl.ds(i * 16, 16)] = (
                ei[r][pl.ds(i * 16, 16)] * (BOND_VOCAB * BOND_VOCAB)
                + ei[r][pl.ds(EB + i * 16, 16)] * BOND_VOCAB
                + ei[r][pl.ds(2 * EB + i * 16, 16)])

    def e_gather_start(r):
        pltpu.async_copy(combo_sp.at[cidx[r]], ebuf[r], gsems[r])

    def e_gather_wait(r):
        pltpu.make_async_copy(combo_sp.at[cidx[r]], ebuf[r], gsems[r]).wait()

    def e_out_start(k, r):
        pltpu.async_copy(ebuf[r], edge_out.at[pl.ds(eblk(k) * EB, EB)], wsems[r])

    def e_out_wait(k, r):
        pltpu.make_async_copy(ebuf[r], edge_out.at[pl.ds(eblk(k) * EB, EB)],
                              wsems[r]).wait()

    def nblk(k):
        return jnp.minimum(wid + k * NW, N_BLOCKS - 1)

    def n_load_start(k, r):
        pltpu.async_copy(xt.at[pl.ds(nblk(k) * NIDX, NIDX)], nidx[r], nisems[r])

    def n_load_wait(k, r):
        pltpu.make_async_copy(xt.at[pl.ds(nblk(k) * NIDX, NIDX)], nidx[r],
                              nisems[r]).wait()

    def n_gather_start(r):
        for f in range(N_ATOM_FEATS):
            pltpu.async_copy(atom_sp.at[nidx[r].at[pl.ds(f * NB, NB)]],
                             nbufs[r][f], nsems[r])

    def n_gather_drain(r):
        for f in range(N_ATOM_FEATS):
            pltpu.make_async_copy(atom_sp.at[nidx[r].at[pl.ds(f * NB, NB)]],
                                  nbufs[r][f], nsems[r]).wait()

    def n_acc(r):
        def acc_loop(n, _):
            for cc in range(EMB_DIM // 16):
                sl = pl.ds(cc * 16, 16)
                v = nbufs[r][0][n, sl]
                for f in range(1, N_ATOM_FEATS):
                    v = v + nbufs[r][f][n, sl]
                nbufs[r][0][n, sl] = v
            return _
        lax.fori_loop(0, NB, acc_loop, 0)

    def n_out_start(k, r):
        pltpu.async_copy(nbufs[r][0], node_out.at[pl.ds(nblk(k) * NB, NB)],
                         nwsems[r])

    def n_out_wait(k, r):
        pltpu.make_async_copy(nbufs[r][0], node_out.at[pl.ds(nblk(k) * NB, NB)],
                              nwsems[r]).wait()

    def n_pair_body(p):
        for b in range(2):
            k = 2 * p + b
            n_load_wait(k, b)
            n_out_wait(k - 2, b)
            n_gather_start(b)
            n_gather_drain(1 - b)
            n_load_start(k + 1, 1 - b)
            n_acc(1 - b)
            n_out_start(k - 1, 1 - b)

    e_load_start(0, 0); e_load_start(1, 1)
    e_load_wait(0, 0); e_cidx(0, 0); e_load_start(2, 0); e_gather_start(0)
    e_load_wait(1, 1); e_cidx(1, 1); e_load_start(3, 1); e_gather_start(1)
    e_gather_wait(0); e_out_start(0, 0)

    n_load_start(0, 0); n_load_start(1, 1)
    n_load_wait(0, 0); n_gather_start(0)
    n_load_wait(1, 1); n_gather_start(1)
    n_gather_drain(0); n_load_start(2, 0); n_acc(0); n_out_start(0, 0)

    def e_pair(g, carry):
        for b in range(2):
            k = 2 * g + b
            e_load_wait(k, b)
            e_cidx(k, b)
            e_load_start(k + 2, b)
            e_out_wait(k - 2, b)
            e_gather_start(b)
            e_gather_wait(1 - b)
            e_out_start(k - 1, 1 - b)

        @pl.when(g % 4 == 0)
        def _do_nodes():
            n_pair_body(g // 4)

        return carry

    lax.fori_loop(1, KE // 2, e_pair, 0)
    e_load_wait(KE, 0); e_load_wait(KE + 1, 1)
    e_gather_wait(1)
    e_out_start(KE - 1, 1)
    e_out_wait(KE - 2, 0)
    e_out_wait(KE - 1, 1)

    n_gather_drain(1); n_acc(1); n_out_start(KN - 1, 1)
    n_out_wait(KN - 2, 0)
    n_out_wait(KN - 1, 1)
    n_load_wait(KN, 0)


@functools.partial(
    pl.kernel,
    out_type=(
        jax.ShapeDtypeStruct((N_NODES, EMB_DIM), jnp.float32),
        jax.ShapeDtypeStruct((N_EDGES, EMB_DIM), jnp.float32),
    ),
    mesh=plsc.VectorSubcoreMesh(core_axis_name="c", subcore_axis_name="s"),
    scratch_types=[
        pltpu.VMEM((N_BOND_FEATS * BOND_VOCAB, EMB_DIM), jnp.float32),
        pltpu.VMEM((COMBO_CHUNK, EMB_DIM), jnp.float32),
        [pltpu.VMEM((N_BOND_FEATS * EB,), jnp.int32) for _ in range(2)],
        [pltpu.VMEM((EB,), jnp.int32) for _ in range(2)],
        [pltpu.VMEM((EB, EMB_DIM), jnp.float32) for _ in range(2)],
        [pltpu.VMEM((NIDX,), jnp.int32) for _ in range(2)],
        [[pltpu.VMEM((NB, EMB_DIM), jnp.float32) for _ in range(N_ATOM_FEATS)]
         for _ in range(2)],
        pltpu.VMEM_SHARED((COMBO_ROWS_PAD, EMB_DIM), jnp.float32),
        pltpu.VMEM_SHARED((ATOM_ROWS_PAD, EMB_DIM), jnp.float32),
        [pltpu.SemaphoreType.DMA for _ in range(2)],
        [pltpu.SemaphoreType.DMA for _ in range(2)],
        [pltpu.SemaphoreType.DMA for _ in range(2)],
        [pltpu.SemaphoreType.DMA for _ in range(2)],
        [pltpu.SemaphoreType.DMA for _ in range(2)],
        [pltpu.SemaphoreType.DMA for _ in range(2)],
    ],
)
def _encoder_sc(xt, et, atomf, bondf, node_out, edge_out, *scratch):
    _sc_body(xt, et, atomf, bondf, node_out, edge_out, *scratch)


def kernel(x, edge_attr, atom_tables, bond_tables):
    xt9 = (x.astype(jnp.int32)
           + (jnp.arange(N_ATOM_FEATS, dtype=jnp.int32) * ATOM_VOCAB)[None, :]).T
    xtb = xt9.reshape(N_ATOM_FEATS, N_BLOCKS, NB).transpose(1, 0, 2).reshape(-1)
    et3 = edge_attr.astype(jnp.int32).T
    etb = et3.reshape(N_BOND_FEATS, E_BLOCKS, EB).transpose(1, 0, 2).reshape(-1)
    atomf = atom_tables.reshape(ATOM_ROWS, EMB_DIM)
    atomf = jnp.pad(atomf, ((0, ATOM_ROWS_PAD - ATOM_ROWS), (0, 0)))
    bondf = bond_tables.reshape(N_BOND_FEATS * BOND_VOCAB, EMB_DIM)
    node_encoded, edge_encoded = _encoder_sc(xtb, etb, atomf, bondf)
    return node_encoded, edge_encoded


# speedup vs baseline: 1.0632x; 1.0632x over previous
import functools

import jax
import jax.numpy as jnp
from jax import lax
from jax.experimental import pallas as pl
from jax.experimental.pallas import tpu as pltpu
from jax.experimental.pallas import tpu_sc as plsc

N_NODES = 10000
N_EDGES = 320000
EMB_DIM = 128
ATOM_VOCAB = 137
BOND_VOCAB = 14
N_ATOM_FEATS = 9
N_BOND_FEATS = 3

NC, NS = 2, 16
NW = NC * NS

ATOM_ROWS = N_ATOM_FEATS * ATOM_VOCAB
ATOM_ROWS_PAD = 1280
ATOM_STAGE = ATOM_ROWS_PAD // NS

COMBO_ROWS = BOND_VOCAB ** 3
COMBO_ROWS_PAD = 2816
COMBO_PER_SUB = COMBO_ROWS_PAD // NS
COMBO_CHUNK = COMBO_PER_SUB // 2

EB = 128
E_BLOCKS = N_EDGES // EB
KE = 80

NB = 16
N_BLOCKS = N_NODES // NB
KN = 20
NIDX = N_ATOM_FEATS * NB


def _sc_body(xt, et, atomf, bondf, node_out, edge_out,
             bond_v, abuf, ei, cidx, ebuf, nidx, nbufs, combo_sp, atom_sp,
             gsems, wsems, nsems, nwsems, isems, nisems):
    c = lax.axis_index("c")
    s = lax.axis_index("s")
    wid = s * NC + c

    pltpu.sync_copy(bondf, bond_v)

    a0 = s * ATOM_STAGE
    pltpu.sync_copy(atomf.at[pl.ds(a0, ATOM_STAGE)], abuf.at[pl.ds(0, ATOM_STAGE)])
    pltpu.sync_copy(abuf.at[pl.ds(0, ATOM_STAGE)], atom_sp.at[pl.ds(a0, ATOM_STAGE)])

    for half in range(2):
        r0 = s * COMBO_PER_SUB + half * COMBO_CHUNK

        def build_row(i, _, r0=r0):
            r = r0 + i
            j0 = r // (BOND_VOCAB * BOND_VOCAB)
            j1 = (r // BOND_VOCAB) % BOND_VOCAB
            j2 = r % BOND_VOCAB
            for cc in range(EMB_DIM // 16):
                sl = pl.ds(cc * 16, 16)
                abuf[i, sl] = (bond_v[j0, sl]
                               + bond_v[BOND_VOCAB + j1, sl]
                               + bond_v[2 * BOND_VOCAB + j2, sl])
            return _

        lax.fori_loop(0, COMBO_CHUNK, build_row, 0)
        pltpu.sync_copy(abuf, combo_sp.at[pl.ds(r0, COMBO_CHUNK)])

    plsc.subcore_barrier()

    def eblk(k):
        return jnp.minimum(wid + k * NW, E_BLOCKS - 1)

    def e_load_start(k, r):
        pltpu.async_copy(et.at[pl.ds(eblk(k) * (N_BOND_FEATS * EB),
                                     N_BOND_FEATS * EB)], ei[r], isems[r])

    def e_load_wait(k, r):
        pltpu.make_async_copy(et.at[pl.ds(eblk(k) * (N_BOND_FEATS * EB),
                                          N_BOND_FEATS * EB)], ei[r],
                              isems[r]).wait()

    def e_cidx(k, r):
        for i in range(EB // 16):
            cidx[r][pl.ds(i * 16, 16)] = (
                ei[r][pl.ds(i * 16, 16)] * (BOND_VOCAB * BOND_VOCAB)
                + ei[r][pl.ds(EB + i * 16, 16)] * BOND_VOCAB
                + ei[r][pl.ds(2 * EB + i * 16, 16)])

    def e_gather_start(r):
        pltpu.async_copy(combo_sp.at[cidx[r]], ebuf[r], gsems[r])

    def e_gather_wait(r):
        pltpu.make_async_copy(combo_sp.at[cidx[r]], ebuf[r], gsems[r]).wait()

    def e_out_start(k, r):
        pltpu.async_copy(ebuf[r], edge_out.at[pl.ds(eblk(k) * EB, EB)], wsems[r])

    def e_out_wait(k, r):
        pltpu.make_async_copy(ebuf[r], edge_out.at[pl.ds(eblk(k) * EB, EB)],
                              wsems[r]).wait()

    def nblk(k):
        return jnp.minimum(wid + k * NW, N_BLOCKS - 1)

    def n_load_start(k, r):
        pltpu.async_copy(xt.at[pl.ds(nblk(k) * NIDX, NIDX)], nidx[r], nisems[r])

    def n_load_wait(k, r):
        pltpu.make_async_copy(xt.at[pl.ds(nblk(k) * NIDX, NIDX)], nidx[r],
                              nisems[r]).wait()

    def n_gather_start(r):
        for f in range(N_ATOM_FEATS):
            pltpu.async_copy(atom_sp.at[nidx[r].at[pl.ds(f * NB, NB)]],
                             nbufs[r][f], nsems[r])

    def n_gather_drain(r):
        for f in range(N_ATOM_FEATS):
            pltpu.make_async_copy(atom_sp.at[nidx[r].at[pl.ds(f * NB, NB)]],
                                  nbufs[r][f], nsems[r]).wait()

    def n_acc(r):
        def acc_loop(n, _):
            for cc in range(EMB_DIM // 16):
                sl = pl.ds(cc * 16, 16)
                v = nbufs[r][0][n, sl]
                for f in range(1, N_ATOM_FEATS):
                    v = v + nbufs[r][f][n, sl]
                nbufs[r][0][n, sl] = v
            return _
        lax.fori_loop(0, NB, acc_loop, 0)

    def n_out_start(k, r):
        pltpu.async_copy(nbufs[r][0], node_out.at[pl.ds(nblk(k) * NB, NB)],
                         nwsems[r])

    def n_out_wait(k, r):
        pltpu.make_async_copy(nbufs[r][0], node_out.at[pl.ds(nblk(k) * NB, NB)],
                              nwsems[r]).wait()

    def n_pair_body(p):
        for b in range(2):
            k = 2 * p + b
            n_load_wait(k, b)
            n_out_wait(k - 2, b)
            n_gather_start(b)
            n_gather_drain(1 - b)
            n_load_start(k + 1, 1 - b)
            n_acc(1 - b)
            n_out_start(k - 1, 1 - b)

    e_load_start(0, 0); e_load_start(1, 1)
    e_load_wait(0, 0); e_cidx(0, 0); e_load_start(2, 0); e_gather_start(0)
    e_load_wait(1, 1); e_cidx(1, 1); e_load_start(3, 1); e_gather_start(1)
    e_gather_wait(0); e_out_start(0, 0)

    def e_pair(g, carry):
        for b in range(2):
            k = 2 * g + b
            e_load_wait(k, b)
            e_cidx(k, b)
            e_load_start(k + 2, b)
            e_out_wait(k - 2, b)
            e_gather_start(b)
            e_gather_wait(1 - b)
            e_out_start(k - 1, 1 - b)
        return carry

    lax.fori_loop(1, KE // 2, e_pair, 0)
    e_load_wait(KE, 0); e_load_wait(KE + 1, 1)
    e_gather_wait(1)
    e_out_start(KE - 1, 1)
    e_out_wait(KE - 2, 0)
    e_out_wait(KE - 1, 1)

    n_load_start(0, 0); n_load_start(1, 1)
    n_load_wait(0, 0); n_gather_start(0)
    n_load_wait(1, 1); n_gather_start(1)
    n_gather_drain(0); n_load_start(2, 0); n_acc(0); n_out_start(0, 0)

    def n_pair(p, carry):
        n_pair_body(p)
        return carry

    lax.fori_loop(1, KN // 2, n_pair, 0)
    n_gather_drain(1); n_acc(1); n_out_start(KN - 1, 1)
    n_out_wait(KN - 2, 0)
    n_out_wait(KN - 1, 1)
    n_load_wait(KN, 0)


@functools.partial(
    pl.kernel,
    out_type=(
        jax.ShapeDtypeStruct((N_NODES, EMB_DIM), jnp.float32),
        jax.ShapeDtypeStruct((N_EDGES, EMB_DIM), jnp.float32),
    ),
    mesh=plsc.VectorSubcoreMesh(core_axis_name="c", subcore_axis_name="s"),
    scratch_types=[
        pltpu.VMEM((N_BOND_FEATS * BOND_VOCAB, EMB_DIM), jnp.float32),
        pltpu.VMEM((COMBO_CHUNK, EMB_DIM), jnp.float32),
        [pltpu.VMEM((N_BOND_FEATS * EB,), jnp.int32) for _ in range(2)],
        [pltpu.VMEM((EB,), jnp.int32) for _ in range(2)],
        [pltpu.VMEM((EB, EMB_DIM), jnp.float32) for _ in range(2)],
        [pltpu.VMEM((NIDX,), jnp.int32) for _ in range(2)],
        [[pltpu.VMEM((NB, EMB_DIM), jnp.float32) for _ in range(N_ATOM_FEATS)]
         for _ in range(2)],
        pltpu.VMEM_SHARED((COMBO_ROWS_PAD, EMB_DIM), jnp.float32),
        pltpu.VMEM_SHARED((ATOM_ROWS_PAD, EMB_DIM), jnp.float32),
        [pltpu.SemaphoreType.DMA for _ in range(2)],
        [pltpu.SemaphoreType.DMA for _ in range(2)],
        [pltpu.SemaphoreType.DMA for _ in range(2)],
        [pltpu.SemaphoreType.DMA for _ in range(2)],
        [pltpu.SemaphoreType.DMA for _ in range(2)],
        [pltpu.SemaphoreType.DMA for _ in range(2)],
    ],
)
def _encoder_sc(xt, et, atomf, bondf, node_out, edge_out, *scratch):
    _sc_body(xt, et, atomf, bondf, node_out, edge_out, *scratch)


def kernel(x, edge_attr, atom_tables, bond_tables):
    xt9 = (x.astype(jnp.int32)
           + (jnp.arange(N_ATOM_FEATS, dtype=jnp.int32) * ATOM_VOCAB)[None, :]).T
    xtb = xt9.reshape(N_ATOM_FEATS, N_BLOCKS, NB).transpose(1, 0, 2).reshape(-1)
    et3 = edge_attr.astype(jnp.int32).T
    etb = et3.reshape(N_BOND_FEATS, E_BLOCKS, EB).transpose(1, 0, 2).reshape(-1)
    atomf = atom_tables.reshape(ATOM_ROWS, EMB_DIM)
    atomf = jnp.pad(atomf, ((0, ATOM_ROWS_PAD - ATOM_ROWS), (0, 0)))
    bondf = bond_tables.reshape(N_BOND_FEATS * BOND_VOCAB, EMB_DIM)
    node_encoded, edge_encoded = _encoder_sc(xtb, etb, atomf, bondf)
    return node_encoded, edge_encoded


# speedup vs baseline: 1.1435x; 1.0755x over previous
import functools

import jax
import jax.numpy as jnp
from jax import lax
from jax.experimental import pallas as pl
from jax.experimental.pallas import tpu as pltpu
from jax.experimental.pallas import tpu_sc as plsc

N_NODES = 10000
N_EDGES = 320000
EMB_DIM = 128
ATOM_VOCAB = 137
BOND_VOCAB = 14
N_ATOM_FEATS = 9
N_BOND_FEATS = 3

NC, NS = 2, 16
NW = NC * NS

ATOM_ROWS = N_ATOM_FEATS * ATOM_VOCAB
ATOM_ROWS_PAD = 1280
ATOM_STAGE = ATOM_ROWS_PAD // NS

COMBO_ROWS = BOND_VOCAB ** 3
COMBO_ROWS_PAD = 2816
COMBO_PER_SUB = COMBO_ROWS_PAD // NS
COMBO_CHUNK = COMBO_PER_SUB // 2

EB = 128
E_BLOCKS = N_EDGES // EB
KE = 78

NB = 16
N_BLOCKS = N_NODES // NB
KN = 18
NIDX = N_ATOM_FEATS * NB


def _sc_body(xt, et, atomf, bondf, node_out, edge_out,
             bond_v, abuf, ei, cidx, ebuf, nidx, nbufs, combo_sp, atom_sp,
             gsems, wsems, nsems, nwsems, isems, nisems):
    c = lax.axis_index("c")
    s = lax.axis_index("s")
    wid = s * NC + c

    pltpu.sync_copy(bondf, bond_v)

    a0 = s * ATOM_STAGE
    pltpu.sync_copy(atomf.at[pl.ds(a0, ATOM_STAGE)], abuf.at[pl.ds(0, ATOM_STAGE)])
    pltpu.sync_copy(abuf.at[pl.ds(0, ATOM_STAGE)], atom_sp.at[pl.ds(a0, ATOM_STAGE)])

    for half in range(2):
        r0 = s * COMBO_PER_SUB + half * COMBO_CHUNK

        def build_row(i, _, r0=r0):
            r = r0 + i
            j0 = r // (BOND_VOCAB * BOND_VOCAB)
            j1 = (r // BOND_VOCAB) % BOND_VOCAB
            j2 = r % BOND_VOCAB
            for cc in range(EMB_DIM // 16):
                sl = pl.ds(cc * 16, 16)
                abuf[i, sl] = (bond_v[j0, sl]
                               + bond_v[BOND_VOCAB + j1, sl]
                               + bond_v[2 * BOND_VOCAB + j2, sl])
            return _

        lax.fori_loop(0, COMBO_CHUNK, build_row, 0)
        pltpu.sync_copy(abuf, combo_sp.at[pl.ds(r0, COMBO_CHUNK)])

    plsc.subcore_barrier()
    plsc.subcore_barrier()

    def eblk(k):
        return jnp.minimum(wid + k * NW, E_BLOCKS - 1)

    def e_load_start(k, r):
        pltpu.async_copy(et.at[pl.ds(eblk(k) * (N_BOND_FEATS * EB),
                                     N_BOND_FEATS * EB)], ei[r], isems[r])

    def e_load_wait(k, r):
        pltpu.make_async_copy(et.at[pl.ds(eblk(k) * (N_BOND_FEATS * EB),
                                          N_BOND_FEATS * EB)], ei[r],
                              isems[r]).wait()

    def e_cidx(k, r):
        for i in range(EB // 16):
            cidx[r][pl.ds(i * 16, 16)] = (
                ei[r][pl.ds(i * 16, 16)] * (BOND_VOCAB * BOND_VOCAB)
                + ei[r][pl.ds(EB + i * 16, 16)] * BOND_VOCAB
                + ei[r][pl.ds(2 * EB + i * 16, 16)])

    def e_gather_start(r):
        pltpu.async_copy(combo_sp.at[cidx[r]], ebuf[r], gsems[r])

    def e_gather_wait(r):
        pltpu.make_async_copy(combo_sp.at[cidx[r]], ebuf[r], gsems[r]).wait()

    def e_out_start(k, r):
        pltpu.async_copy(ebuf[r], edge_out.at[pl.ds(eblk(k) * EB, EB)], wsems[r])

    def e_out_wait(k, r):
        pltpu.make_async_copy(ebuf[r], edge_out.at[pl.ds(eblk(k) * EB, EB)],
                              wsems[r]).wait()

    def nblk(k):
        return jnp.minimum(wid + k * NW, N_BLOCKS - 1)

    def n_load_start(k, r):
        pltpu.async_copy(xt.at[pl.ds(nblk(k) * NIDX, NIDX)], nidx[r], nisems[r])

    def n_load_wait(k, r):
        pltpu.make_async_copy(xt.at[pl.ds(nblk(k) * NIDX, NIDX)], nidx[r],
                              nisems[r]).wait()

    def n_gather_start(r):
        for f in range(N_ATOM_FEATS):
            pltpu.async_copy(atom_sp.at[nidx[r].at[pl.ds(f * NB, NB)]],
                             nbufs[r][f], nsems[r])

    def n_gather_drain(r):
        for f in range(N_ATOM_FEATS):
            pltpu.make_async_copy(atom_sp.at[nidx[r].at[pl.ds(f * NB, NB)]],
                                  nbufs[r][f], nsems[r]).wait()

    def n_acc(r):
        def acc_loop(n, _):
            for cc in range(EMB_DIM // 16):
                sl = pl.ds(cc * 16, 16)
                v = nbufs[r][0][n, sl]
                for f in range(1, N_ATOM_FEATS):
                    v = v + nbufs[r][f][n, sl]
                nbufs[r][0][n, sl] = v
            return _
        lax.fori_loop(0, NB, acc_loop, 0)

    def n_out_start(k, r):
        pltpu.async_copy(nbufs[r][0], node_out.at[pl.ds(nblk(k) * NB, NB)],
                         nwsems[r])

    def n_out_wait(k, r):
        pltpu.make_async_copy(nbufs[r][0], node_out.at[pl.ds(nblk(k) * NB, NB)],
                              nwsems[r]).wait()

    def n_pair_body(p):
        for b in range(2):
            k = 2 * p + b
            n_load_wait(k, b)
            n_out_wait(k - 2, b)
            n_gather_start(b)
            n_gather_drain(1 - b)
            n_load_start(k + 1, 1 - b)
            n_acc(1 - b)
            n_out_start(k - 1, 1 - b)

    e_load_start(0, 0); e_load_start(1, 1)
    e_load_wait(0, 0); e_cidx(0, 0); e_load_start(2, 0); e_gather_start(0)
    e_load_wait(1, 1); e_cidx(1, 1); e_load_start(3, 1); e_gather_start(1)
    e_gather_wait(0); e_out_start(0, 0)

    def e_pair(g, carry):
        for b in range(2):
            k = 2 * g + b
            e_load_wait(k, b)
            e_cidx(k, b)
            e_load_start(k + 2, b)
            e_out_wait(k - 2, b)
            e_gather_start(b)
            e_gather_wait(1 - b)
            e_out_start(k - 1, 1 - b)
        return carry

    lax.fori_loop(1, KE // 2, e_pair, 0)
    e_load_wait(KE, 0)
    e_cidx(KE, 0)
    e_out_wait(KE - 2, 0)
    e_gather_start(0)
    e_gather_wait(1)
    e_out_start(KE - 1, 1)
    e_load_wait(KE + 1, 1)
    e_gather_wait(0)

    @pl.when(wid < E_BLOCKS - KE * NW)
    def _edge_tail_write():
        pltpu.async_copy(ebuf[0], edge_out.at[pl.ds(eblk(KE) * EB, EB)],
                         wsems[0])
        pltpu.make_async_copy(ebuf[0], edge_out.at[pl.ds(eblk(KE) * EB, EB)],
                              wsems[0]).wait()

    e_out_wait(KE - 1, 1)

    n_load_start(0, 0); n_load_start(1, 1)
    n_load_wait(0, 0); n_gather_start(0)
    n_load_wait(1, 1); n_gather_start(1)
    n_gather_drain(0); n_load_start(2, 0); n_acc(0); n_out_start(0, 0)

    def n_pair(p, carry):
        n_pair_body(p)
        return carry

    lax.fori_loop(1, KN // 2, n_pair, 0)
    n_load_wait(KN, 0)
    n_gather_drain(1)
    n_load_start(KN + 1, 1)
    n_acc(1); n_out_start(KN - 1, 1)
    n_out_wait(KN - 2, 0)
    n_gather_start(0)
    n_load_wait(KN + 1, 1)
    n_gather_drain(0)
    n_acc(0)
    n_out_start(KN, 0)
    n_out_wait(KN - 1, 1)
    n_gather_start(1)
    n_gather_drain(1)
    n_acc(1)

    @pl.when(wid < N_BLOCKS - (KN + 1) * NW)
    def _node_tail_write():
        pltpu.async_copy(nbufs[1][0],
                         node_out.at[pl.ds(nblk(KN + 1) * NB, NB)], nwsems[1])
        pltpu.make_async_copy(nbufs[1][0],
                              node_out.at[pl.ds(nblk(KN + 1) * NB, NB)],
                              nwsems[1]).wait()

    n_out_wait(KN, 0)


@functools.partial(
    pl.kernel,
    out_type=(
        jax.ShapeDtypeStruct((N_NODES, EMB_DIM), jnp.float32),
        jax.ShapeDtypeStruct((N_EDGES, EMB_DIM), jnp.float32),
    ),
    mesh=plsc.VectorSubcoreMesh(core_axis_name="c", subcore_axis_name="s"),
    scratch_types=[
        pltpu.VMEM((N_BOND_FEATS * BOND_VOCAB, EMB_DIM), jnp.float32),
        pltpu.VMEM((COMBO_CHUNK, EMB_DIM), jnp.float32),
        [pltpu.VMEM((N_BOND_FEATS * EB,), jnp.int32) for _ in range(2)],
        [pltpu.VMEM((EB,), jnp.int32) for _ in range(2)],
        [pltpu.VMEM((EB, EMB_DIM), jnp.float32) for _ in range(2)],
        [pltpu.VMEM((NIDX,), jnp.int32) for _ in range(2)],
        [[pltpu.VMEM((NB, EMB_DIM), jnp.float32) for _ in range(N_ATOM_FEATS)]
         for _ in range(2)],
        pltpu.VMEM_SHARED((COMBO_ROWS_PAD, EMB_DIM), jnp.float32),
        pltpu.VMEM_SHARED((ATOM_ROWS_PAD, EMB_DIM), jnp.float32),
        [pltpu.SemaphoreType.DMA for _ in range(2)],
        [pltpu.SemaphoreType.DMA for _ in range(2)],
        [pltpu.SemaphoreType.DMA for _ in range(2)],
        [pltpu.SemaphoreType.DMA for _ in range(2)],
        [pltpu.SemaphoreType.DMA for _ in range(2)],
        [pltpu.SemaphoreType.DMA for _ in range(2)],
    ],
)
def _encoder_sc(xt, et, atomf, bondf, node_out, edge_out, *scratch):
    _sc_body(xt, et, atomf, bondf, node_out, edge_out, *scratch)


def kernel(x, edge_attr, atom_tables, bond_tables):
    xt9 = (x.astype(jnp.int32)
           + (jnp.arange(N_ATOM_FEATS, dtype=jnp.int32) * ATOM_VOCAB)[None, :]).T
    xtb = xt9.reshape(N_ATOM_FEATS, N_BLOCKS, NB).transpose(1, 0, 2).reshape(-1)
    et3 = edge_attr.astype(jnp.int32).T
    etb = et3.reshape(N_BOND_FEATS, E_BLOCKS, EB).transpose(1, 0, 2).reshape(-1)
    atomf = atom_tables.reshape(ATOM_ROWS, EMB_DIM)
    atomf = jnp.pad(atomf, ((0, ATOM_ROWS_PAD - ATOM_ROWS), (0, 0)))
    bondf = bond_tables.reshape(N_BOND_FEATS * BOND_VOCAB, EMB_DIM)
    node_encoded, edge_encoded = _encoder_sc(xtb, etb, atomf, bondf)
    return node_encoded, edge_encoded
